# Initial kernel scaffold; baseline (speedup 1.0000x reference)
import jax
import jax.numpy as jnp
from jax.experimental import pallas as pl


def kernel(classifications, regressions, anchors, annotations, image_names):
    raise NotImplementedError("write your pallas kernel here")


# speedup vs baseline: 7.4757x; 7.4757x over previous
import functools

import jax
import jax.numpy as jnp
from jax.experimental import pallas as pl
from jax.experimental.pallas import tpu as pltpu

_ALPHA = 0.25
_POS_THR = 0.5
_NEG_THR = 0.4
_BETA = 1.0 / 9

_B, _N, _C, _M = 8, 20000, 80, 50
_NP = 20480
_K = 2048
_NCHUNK = _NP // _K


def _body(cls_ref, reg_ref, anc_ref, ann_ref, annT_ref, outc_ref, outr_ref,
          ov_scr):
    b = pl.program_id(0)

    ann = ann_ref[0]
    annT = annT_ref[0]
    gx1 = ann[:, 0:1]
    gy1 = ann[:, 1:2]
    gx2 = ann[:, 2:3]
    gy2 = ann[:, 3:4]
    area_g = (gx2 - gx1) * (gy2 - gy1)

    iota_m = jax.lax.broadcasted_iota(jnp.int32, (_M, _K), 0)
    iota_c = jax.lax.broadcasted_iota(jnp.int32, (_C, _K), 0)
    lane_i = jax.lax.broadcasted_iota(jnp.int32, (1, _K), 1)

    gt_max = jnp.full((_M, 1), -1.0, dtype=jnp.float32)
    for c in range(_NCHUNK):
        a = anc_ref[:, c * _K:(c + 1) * _K]
        ax1 = a[0:1, :]
        ay1 = a[1:2, :]
        ax2 = a[2:3, :]
        ay2 = a[3:4, :]
        area_a = (ax2 - ax1) * (ay2 - ay1)
        ltx = jnp.maximum(ax1, gx1)
        lty = jnp.maximum(ay1, gy1)
        rbx = jnp.minimum(ax2, gx2)
        rby = jnp.minimum(ay2, gy2)
        whx = jnp.maximum(rbx - ltx, 0.0)
        why = jnp.maximum(rby - lty, 0.0)
        inter = whx * why
        union = area_a + area_g - inter
        ov = inter / jnp.maximum(union, 1e-6)
        ov_scr[:, c * _K:(c + 1) * _K] = ov
        gt_max = jnp.maximum(gt_max, jnp.max(ov, axis=1, keepdims=True))

    cls_acc = jnp.float32(0.0)
    reg_acc = jnp.float32(0.0)
    np_acc = jnp.float32(0.0)
    for c in range(_NCHUNK):
        ov = ov_scr[:, c * _K:(c + 1) * _K]
        maxov = jnp.max(ov, axis=0, keepdims=True)
        eq = ov == maxov
        amax = jnp.min(jnp.where(eq, iota_m, _M), axis=0, keepdims=True)
        lq = jnp.max((ov == gt_max).astype(jnp.float32), axis=0,
                     keepdims=True) > 0.0
        w = ((lane_i + c * _K) < _N)
        pos05 = maxov >= _POS_THR
        basef = ((pos05 | (maxov < _NEG_THR)) & w).astype(jnp.float32)
        posf = ((pos05 | lq) & w).astype(jnp.float32)

        onehot_m = (iota_m == amax).astype(jnp.float32)
        assigned = jax.lax.dot_general(
            annT, onehot_m, (((1,), (0,)), ((), ())),
            preferred_element_type=jnp.float32)
        label = assigned[4:5, :].astype(jnp.int32)

        p = jnp.clip(cls_ref[0, :, c * _K:(c + 1) * _K], 1e-4, 1.0 - 1e-4)
        l0 = (0.75 * (p * p)) * (-jnp.log(1.0 - p + 1e-6))
        s_bg = jnp.sum(l0, axis=0, keepdims=True)
        sel = jnp.sum(jnp.where(iota_c == label, p, 0.0), axis=0,
                      keepdims=True)
        l0_l = (0.75 * (sel * sel)) * (-jnp.log(1.0 - sel + 1e-6))
        oms = 1.0 - sel
        l1_l = (0.25 * (oms * oms)) * (-jnp.log(sel + 1e-6))
        cls_acc += jnp.sum(basef * s_bg + posf * l1_l - (posf * basef) * l0_l)
        np_acc += jnp.sum(posf)

        a = anc_ref[:, c * _K:(c + 1) * _K]
        aw = a[2:3, :] - a[0:1, :]
        ah = a[3:4, :] - a[1:2, :]
        axc = a[0:1, :] + 0.5 * aw
        ayc = a[1:2, :] + 0.5 * ah
        gw = assigned[2:3, :] - assigned[0:1, :]
        gh = assigned[3:4, :] - assigned[1:2, :]
        gxc = assigned[0:1, :] + 0.5 * gw
        gyc = assigned[1:2, :] + 0.5 * gh
        r = reg_ref[0, :, c * _K:(c + 1) * _K]
        d0 = jnp.abs(r[0:1, :] - (gxc - axc) / aw)
        d1 = jnp.abs(r[1:2, :] - (gyc - ayc) / ah)
        d2 = jnp.abs(r[2:3, :] - jnp.log(gw / aw))
        d3 = jnp.abs(r[3:4, :] - jnp.log(gh / ah))

        def _sl1(d):
            return jnp.where(d < _BETA, 0.5 * d * d / _BETA, d - 0.5 * _BETA)

        reg_acc += jnp.sum(posf * (_sl1(d0) + _sl1(d1) + _sl1(d2) + _sl1(d3)))

    cls_img = cls_acc / jnp.maximum(np_acc, 1.0)
    reg_img = jnp.where(np_acc > 0.0,
                        reg_acc / jnp.maximum(np_acc * 4.0, 1.0), 0.0)

    cls_v = jnp.reshape(cls_img * 0.125, (1, 1))
    reg_v = jnp.reshape(reg_img * 0.125, (1, 1))

    @pl.when(b == 0)
    def _():
        outc_ref[:, :] = cls_v
        outr_ref[:, :] = reg_v

    @pl.when(b != 0)
    def _():
        outc_ref[:, :] += cls_v
        outr_ref[:, :] += reg_v


@functools.partial(jax.jit, static_argnames=("interpret",))
def _run(classifications, regressions, anchors, annotations, interpret=False):
    pad = _NP - _N
    clsT = jnp.pad(jnp.transpose(classifications, (0, 2, 1)),
                   ((0, 0), (0, 0), (0, pad)), constant_values=0.5)
    regT = jnp.pad(jnp.transpose(regressions, (0, 2, 1)),
                   ((0, 0), (0, 0), (0, pad)))
    ancT = jnp.transpose(anchors[0])
    far = jnp.tile(jnp.array([[1e9], [1e9], [2e9], [2e9]], jnp.float32),
                   (1, pad))
    ancT = jnp.concatenate([ancT, far], axis=1)
    annT = jnp.transpose(annotations, (0, 2, 1))

    outc, outr = pl.pallas_call(
        _body,
        grid=(_B,),
        in_specs=[
            pl.BlockSpec((1, _C, _NP), lambda b: (b, 0, 0)),
            pl.BlockSpec((1, 4, _NP), lambda b: (b, 0, 0)),
            pl.BlockSpec((4, _NP), lambda b: (0, 0)),
            pl.BlockSpec((1, _M, 5), lambda b: (b, 0, 0)),
            pl.BlockSpec((1, 5, _M), lambda b: (b, 0, 0)),
        ],
        out_specs=[
            pl.BlockSpec((1, 1), lambda b: (0, 0)),
            pl.BlockSpec((1, 1), lambda b: (0, 0)),
        ],
        out_shape=[jax.ShapeDtypeStruct((1, 1), jnp.float32)] * 2,
        scratch_shapes=[pltpu.VMEM((_M, _NP), jnp.float32)],
        interpret=interpret,
    )(clsT, regT, ancT, annotations, annT)
    return outc.reshape(1), outr.reshape(1)


def kernel(classifications, regressions, anchors, annotations, image_names):
    del image_names
    return _run(classifications, regressions, anchors, annotations)
